# Initial kernel scaffold; baseline (speedup 1.0000x reference)
import jax
import jax.numpy as jnp
from jax.experimental import pallas as pl


def kernel(x, pos, batch, W1, b1, W2, b2):
    raise NotImplementedError("write your pallas kernel here")


# speedup vs baseline: 8.2600x; 8.2600x over previous
import functools

import jax
import jax.numpy as jnp
from jax import lax
from jax.experimental import pallas as pl
from jax.experimental.pallas import tpu as pltpu
from jax.experimental.pallas import tpu_sc as plsc

N = 10000
D = 128
K = 64
H = 128
R2 = 0.12 * 0.12

NW = 32
QPW = 320
NP = NW * QPW
NV = NP // 16
BQ = 64
PAD_POS = 1e9


def _prep_body(x_ref, pos_ref, w1a_ref, w1b_ref, b1_ref, s_ref, q_ref):
    pw = (pos_ref[:, 0:1] * w1b_ref[0:1, :]
          + pos_ref[:, 1:2] * w1b_ref[1:2, :]
          + pos_ref[:, 2:3] * w1b_ref[2:3, :])
    xw = lax.dot_general(x_ref[...], w1a_ref[...], (((1,), (0,)), ((), ())),
                         preferred_element_type=jnp.float32)
    s_ref[...] = xw + pw
    q_ref[...] = b1_ref[...] - pw


def _prep(xp, posp, w1a, w1b, b1r):
    blk = 256
    return pl.pallas_call(
        _prep_body,
        grid=(NP // blk,),
        in_specs=[
            pl.BlockSpec((blk, D), lambda i: (i, 0)),
            pl.BlockSpec((blk, 3), lambda i: (i, 0)),
            pl.BlockSpec((D, H), lambda i: (0, 0)),
            pl.BlockSpec((3, H), lambda i: (0, 0)),
            pl.BlockSpec((1, H), lambda i: (0, 0)),
        ],
        out_specs=[
            pl.BlockSpec((blk, H), lambda i: (i, 0)),
            pl.BlockSpec((blk, H), lambda i: (i, 0)),
        ],
        out_shape=[
            jax.ShapeDtypeStruct((NP, H), jnp.float32),
            jax.ShapeDtypeStruct((NP, H), jnp.float32),
        ],
    )(xp, posp, w1a, w1b, b1r)


_mesh = plsc.VectorSubcoreMesh(core_axis_name="c", subcore_axis_name="s")


@functools.partial(
    pl.kernel,
    mesh=_mesh,
    out_type=[
        jax.ShapeDtypeStruct((NP * K, H), jnp.float32),
        jax.ShapeDtypeStruct((NP,), jnp.int32),
    ],
    scratch_types=[
        pltpu.VMEM((NP,), jnp.float32),
        pltpu.VMEM((NP,), jnp.float32),
        pltpu.VMEM((NP,), jnp.float32),
        pltpu.VMEM((NP,), jnp.float32),
        pltpu.VMEM((NP + 32,), jnp.float32),
        pltpu.VMEM((NP + 32,), jnp.int32),
        pltpu.VMEM((QPW * K,), jnp.int32),
        pltpu.VMEM((QPW,), jnp.int32),
        pltpu.VMEM((128, H), jnp.float32),
        pltpu.SemaphoreType.DMA,
    ],
    compiler_params=pltpu.CompilerParams(needs_layout_passes=False),
)
def _search(posx_hbm, posy_hbm, posz_hbm, posxb_hbm, posyb_hbm, poszb_hbm,
            s_hbm, sg_hbm, cnt_hbm, px, py, pz, sq, cd, ci, nbrs, cnts,
            rows, sem):
    cid = lax.axis_index("c")
    sid = lax.axis_index("s")
    wid = sid * 2 + cid
    q0 = wid * QPW

    pltpu.sync_copy(posx_hbm, px)
    pltpu.sync_copy(posy_hbm, py)
    pltpu.sync_copy(posz_hbm, pz)

    lanes = lax.iota(jnp.int32, 16)
    lane0 = lanes == 0
    inf16 = jnp.full((16,), jnp.inf, jnp.float32)

    def sq4(v4, carry):
        for u in range(4):
            base = (v4 * 4 + u) * 16
            xv = px[pl.ds(base, 16)]
            yv = py[pl.ds(base, 16)]
            zv = pz[pl.ds(base, 16)]
            sq[pl.ds(base, 16)] = (xv * xv + yv * yv) + zv * zv
        return carry

    lax.fori_loop(0, NV // 4, sq4, jnp.int32(0))

    pltpu.sync_copy(posxb_hbm, px)
    pltpu.sync_copy(posyb_hbm, py)
    pltpu.sync_copy(poszb_hbm, pz)

    def per_query(qi, carry):
        q = q0 + qi
        qs = jnp.full((16,), q, jnp.int32)
        qx = plsc.load_gather(px, [qs])
        qy = plsc.load_gather(py, [qs])
        qz = plsc.load_gather(pz, [qs])
        qsq = plsc.load_gather(sq, [qs])

        def scan4(v4, off):
            for u in range(4):
                base = (v4 * 4 + u) * 16
                dot = (qx * px[pl.ds(base, 16)]
                       + qy * py[pl.ds(base, 16)]) + qz * pz[pl.ds(base, 16)]
                d2 = (qsq + sq[pl.ds(base, 16)]) - 2.0 * dot
                m = d2 <= R2
                plsc.store_compressed(cd.at[pl.ds(off, 16)], d2, mask=m)
                plsc.store_compressed(ci.at[pl.ds(off, 16)], base + lanes, mask=m)
                off = off + jnp.sum(m.astype(jnp.int32))
            return off

        C = lax.fori_loop(0, NV // 4, scan4, jnp.int32(0))
        cd[pl.ds(C, 16)] = inf16
        cnt = jnp.minimum(C, K)
        vc = (C + 15) // 16

        plsc.store_scatter(nbrs, [jnp.full((16,), qi * K, jnp.int32)], qs,
                           mask=lane0)

        def extract(k, _):
            def m1(v, acc):
                return jnp.minimum(acc, cd[pl.ds(v * 16, 16)])

            mv = lax.fori_loop(0, vc, m1, inf16)
            mn = jnp.min(mv)

            def m2(v, acc):
                d = cd[pl.ds(v * 16, 16)]
                p = v * 16 + lanes
                return jnp.minimum(acc, jnp.where(d == mn, p, NP * 4))

            pv = lax.fori_loop(0, vc, m2, jnp.full((16,), NP * 4, jnp.int32))
            p = jnp.min(pv)
            ps = jnp.full((16,), p, jnp.int32)
            nv = plsc.load_gather(ci, [ps])
            plsc.store_scatter(nbrs, [jnp.full((16,), qi * K + k, jnp.int32)],
                               nv, mask=lane0)
            plsc.store_scatter(cd, [ps], inf16, mask=lane0)
            return _

        lax.fori_loop(0, cnt, extract, jnp.int32(0))

        nbr0 = plsc.load_gather(nbrs, [jnp.full((16,), qi * K, jnp.int32)])
        for t in range(4):
            kvec = t * 16 + lanes
            plsc.store_scatter(nbrs, [qi * K + kvec], nbr0, mask=kvec >= cnt)
        plsc.store_scatter(cnts, [jnp.full((16,), qi, jnp.int32)],
                           jnp.full((16,), cnt, jnp.int32), mask=lane0)
        return carry

    lax.fori_loop(0, QPW, per_query, jnp.int32(0))
    pltpu.sync_copy(cnts, cnt_hbm.at[pl.ds(q0, QPW)])

    row0 = q0 * K

    def gchunk(ch, carry):
        idx = nbrs.at[pl.ds(ch * 128, 128)]
        pltpu.async_copy(s_hbm.at[idx], rows, sem).wait()
        pltpu.sync_copy(rows, sg_hbm.at[pl.ds(row0 + ch * 128, 128)])
        return carry

    lax.fori_loop(0, QPW * K // 128, gchunk, jnp.int32(0))


def _conv_body(sg_ref, q_ref, w2_ref, b2_ref, o_ref):
    z3 = sg_ref[...].reshape(BQ, K, H) + q_ref[...][:, None, :]
    z = jnp.maximum(z3, 0.0).reshape(BQ * K, H)
    h = lax.dot_general(z, w2_ref[...], (((1,), (0,)), ((), ())),
                        preferred_element_type=jnp.float32) + b2_ref[...]
    o_ref[...] = jnp.max(h.reshape(BQ, K, H), axis=1)


def _conv(sg, q, W2, b2r):
    return pl.pallas_call(
        _conv_body,
        grid=(NP // BQ,),
        in_specs=[
            pl.BlockSpec((BQ * K, H), lambda i: (i, 0)),
            pl.BlockSpec((BQ, H), lambda i: (i, 0)),
            pl.BlockSpec((H, H), lambda i: (0, 0)),
            pl.BlockSpec((1, H), lambda i: (0, 0)),
        ],
        out_specs=pl.BlockSpec((BQ, H), lambda i: (i, 0)),
        out_shape=jax.ShapeDtypeStruct((NP, H), jnp.float32),
    )(sg, q, W2, b2r)


def kernel(x, pos, batch, W1, b1, W2, b2):
    pad = NP - N
    xp = jnp.concatenate([x, jnp.zeros((pad, D), jnp.float32)], axis=0)
    posp = jnp.concatenate(
        [pos, jnp.full((pad, 3), PAD_POS, jnp.float32)], axis=0)
    w1a = W1[:D]
    w1b = W1[D:]
    b1r = b1.reshape(1, H)
    b2r = b2.reshape(1, H)

    posb = lax.reduce_precision(posp, exponent_bits=8, mantissa_bits=7)
    s, q = _prep(xp, posp, w1a, w1b, b1r)
    sg, cnt = _search(posp[:, 0], posp[:, 1], posp[:, 2],
                      posb[:, 0], posb[:, 1], posb[:, 2], s)
    outp = _conv(sg, q, W2, b2r)
    out = jnp.where(cnt[:N, None] > 0, outp[:N], jnp.float32(-1e30))
    return (out, pos, batch)


# speedup vs baseline: 8.5125x; 1.0306x over previous
import functools

import jax
import jax.numpy as jnp
from jax import lax
from jax.experimental import pallas as pl
from jax.experimental.pallas import tpu as pltpu
from jax.experimental.pallas import tpu_sc as plsc

N = 10000
D = 128
K = 64
H = 128
R2 = 0.12 * 0.12

NW = 32
QPW = 320
NP = NW * QPW
NV = NP // 16
BQ = 64
PAD_POS = 1e9


def _prep_body(x_ref, pos_ref, w1a_ref, w1b_ref, b1_ref, s_ref, q_ref):
    pw = (pos_ref[:, 0:1] * w1b_ref[0:1, :]
          + pos_ref[:, 1:2] * w1b_ref[1:2, :]
          + pos_ref[:, 2:3] * w1b_ref[2:3, :])
    xw = lax.dot_general(x_ref[...], w1a_ref[...], (((1,), (0,)), ((), ())),
                         preferred_element_type=jnp.float32)
    s_ref[...] = xw + pw
    q_ref[...] = b1_ref[...] - pw


def _prep(xp, posp, w1a, w1b, b1r):
    blk = 256
    return pl.pallas_call(
        _prep_body,
        grid=(NP // blk,),
        in_specs=[
            pl.BlockSpec((blk, D), lambda i: (i, 0)),
            pl.BlockSpec((blk, 3), lambda i: (i, 0)),
            pl.BlockSpec((D, H), lambda i: (0, 0)),
            pl.BlockSpec((3, H), lambda i: (0, 0)),
            pl.BlockSpec((1, H), lambda i: (0, 0)),
        ],
        out_specs=[
            pl.BlockSpec((blk, H), lambda i: (i, 0)),
            pl.BlockSpec((blk, H), lambda i: (i, 0)),
        ],
        out_shape=[
            jax.ShapeDtypeStruct((NP, H), jnp.float32),
            jax.ShapeDtypeStruct((NP, H), jnp.float32),
        ],
    )(xp, posp, w1a, w1b, b1r)


_mesh = plsc.VectorSubcoreMesh(core_axis_name="c", subcore_axis_name="s")


@functools.partial(
    pl.kernel,
    mesh=_mesh,
    out_type=[
        jax.ShapeDtypeStruct((NP * K, H), jnp.float32),
        jax.ShapeDtypeStruct((NP,), jnp.int32),
    ],
    scratch_types=[
        pltpu.VMEM((NP,), jnp.float32),
        pltpu.VMEM((NP,), jnp.float32),
        pltpu.VMEM((NP,), jnp.float32),
        pltpu.VMEM((NP,), jnp.float32),
        pltpu.VMEM((NP + 32,), jnp.float32),
        pltpu.VMEM((NP + 32,), jnp.int32),
        pltpu.VMEM((QPW * K,), jnp.int32),
        pltpu.VMEM((QPW,), jnp.int32),
        pltpu.VMEM((128, H), jnp.float32),
        pltpu.VMEM((128, H), jnp.float32),
        pltpu.SemaphoreType.DMA,
        pltpu.SemaphoreType.DMA,
        pltpu.SemaphoreType.DMA,
        pltpu.SemaphoreType.DMA,
    ],
    compiler_params=pltpu.CompilerParams(needs_layout_passes=False),
)
def _search(posx_hbm, posy_hbm, posz_hbm, posxb_hbm, posyb_hbm, poszb_hbm,
            s_hbm, sg_hbm, cnt_hbm, px, py, pz, sq, cd, ci, nbrs, cnts,
            rows0, rows1, semg0, semg1, semw0, semw1):
    cid = lax.axis_index("c")
    sid = lax.axis_index("s")
    wid = sid * 2 + cid
    q0 = wid * QPW

    pltpu.sync_copy(posx_hbm, px)
    pltpu.sync_copy(posy_hbm, py)
    pltpu.sync_copy(posz_hbm, pz)

    lanes = lax.iota(jnp.int32, 16)
    lane0 = lanes == 0
    inf16 = jnp.full((16,), jnp.inf, jnp.float32)

    def sq4(v4, carry):
        for u in range(4):
            base = (v4 * 4 + u) * 16
            xv = px[pl.ds(base, 16)]
            yv = py[pl.ds(base, 16)]
            zv = pz[pl.ds(base, 16)]
            sq[pl.ds(base, 16)] = (xv * xv + yv * yv) + zv * zv
        return carry

    lax.fori_loop(0, NV // 4, sq4, jnp.int32(0))

    pltpu.sync_copy(posxb_hbm, px)
    pltpu.sync_copy(posyb_hbm, py)
    pltpu.sync_copy(poszb_hbm, pz)

    def per_query(qi, carry):
        q = q0 + qi
        qs = jnp.full((16,), q, jnp.int32)
        qx = plsc.load_gather(px, [qs])
        qy = plsc.load_gather(py, [qs])
        qz = plsc.load_gather(pz, [qs])
        qsq = plsc.load_gather(sq, [qs])

        def scan4(v4, off):
            for u in range(4):
                base = (v4 * 4 + u) * 16
                dot = (qx * px[pl.ds(base, 16)]
                       + qy * py[pl.ds(base, 16)]) + qz * pz[pl.ds(base, 16)]
                d2 = (qsq + sq[pl.ds(base, 16)]) - 2.0 * dot
                m = d2 <= R2
                plsc.store_compressed(cd.at[pl.ds(off, 16)], d2, mask=m)
                plsc.store_compressed(ci.at[pl.ds(off, 16)], base + lanes, mask=m)
                off = off + jnp.sum(m.astype(jnp.int32))
            return off

        C = lax.fori_loop(0, NV // 4, scan4, jnp.int32(0))
        cd[pl.ds(C, 16)] = inf16
        cnt = jnp.minimum(C, K)
        vc = (C + 15) // 16

        plsc.store_scatter(nbrs, [jnp.full((16,), qi * K, jnp.int32)], qs,
                           mask=lane0)

        def extract(k, _):
            def m1(v, acc):
                return jnp.minimum(acc, cd[pl.ds(v * 16, 16)])

            mv = lax.fori_loop(0, vc, m1, inf16)
            mn = jnp.min(mv)

            def m2(v, acc):
                d = cd[pl.ds(v * 16, 16)]
                p = v * 16 + lanes
                return jnp.minimum(acc, jnp.where(d == mn, p, NP * 4))

            pv = lax.fori_loop(0, vc, m2, jnp.full((16,), NP * 4, jnp.int32))
            p = jnp.min(pv)
            ps = jnp.full((16,), p, jnp.int32)
            nv = plsc.load_gather(ci, [ps])
            plsc.store_scatter(nbrs, [jnp.full((16,), qi * K + k, jnp.int32)],
                               nv, mask=lane0)
            plsc.store_scatter(cd, [ps], inf16, mask=lane0)
            return _

        lax.fori_loop(0, cnt, extract, jnp.int32(0))

        nbr0 = plsc.load_gather(nbrs, [jnp.full((16,), qi * K, jnp.int32)])
        for t in range(4):
            kvec = t * 16 + lanes
            plsc.store_scatter(nbrs, [qi * K + kvec], nbr0, mask=kvec >= cnt)
        plsc.store_scatter(cnts, [jnp.full((16,), qi, jnp.int32)],
                           jnp.full((16,), cnt, jnp.int32), mask=lane0)
        return carry

    lax.fori_loop(0, QPW, per_query, jnp.int32(0))
    pltpu.sync_copy(cnts, cnt_hbm.at[pl.ds(q0, QPW)])

    row0 = q0 * K

    def gpair(p, carry):
        ch0 = 2 * p
        ga = pltpu.async_copy(
            s_hbm.at[nbrs.at[pl.ds(ch0 * 128, 128)]], rows0, semg0)
        gb = pltpu.async_copy(
            s_hbm.at[nbrs.at[pl.ds((ch0 + 1) * 128, 128)]], rows1, semg1)
        ga.wait()
        wa = pltpu.async_copy(
            rows0, sg_hbm.at[pl.ds(row0 + ch0 * 128, 128)], semw0)
        gb.wait()
        wb = pltpu.async_copy(
            rows1, sg_hbm.at[pl.ds(row0 + (ch0 + 1) * 128, 128)], semw1)
        wa.wait()
        wb.wait()
        return carry

    lax.fori_loop(0, QPW * K // 256, gpair, jnp.int32(0))


def _conv_body(sg_ref, q_ref, w2_ref, b2_ref, o_ref):
    z3 = sg_ref[...].reshape(BQ, K, H) + q_ref[...][:, None, :]
    z = jnp.maximum(z3, 0.0).reshape(BQ * K, H)
    h = lax.dot_general(z, w2_ref[...], (((1,), (0,)), ((), ())),
                        preferred_element_type=jnp.float32) + b2_ref[...]
    o_ref[...] = jnp.max(h.reshape(BQ, K, H), axis=1)


def _conv(sg, q, W2, b2r):
    return pl.pallas_call(
        _conv_body,
        grid=(NP // BQ,),
        in_specs=[
            pl.BlockSpec((BQ * K, H), lambda i: (i, 0)),
            pl.BlockSpec((BQ, H), lambda i: (i, 0)),
            pl.BlockSpec((H, H), lambda i: (0, 0)),
            pl.BlockSpec((1, H), lambda i: (0, 0)),
        ],
        out_specs=pl.BlockSpec((BQ, H), lambda i: (i, 0)),
        out_shape=jax.ShapeDtypeStruct((NP, H), jnp.float32),
    )(sg, q, W2, b2r)


def kernel(x, pos, batch, W1, b1, W2, b2):
    pad = NP - N
    xp = jnp.concatenate([x, jnp.zeros((pad, D), jnp.float32)], axis=0)
    posp = jnp.concatenate(
        [pos, jnp.full((pad, 3), PAD_POS, jnp.float32)], axis=0)
    w1a = W1[:D]
    w1b = W1[D:]
    b1r = b1.reshape(1, H)
    b2r = b2.reshape(1, H)

    posb = lax.reduce_precision(posp, exponent_bits=8, mantissa_bits=7)
    s, q = _prep(xp, posp, w1a, w1b, b1r)
    sg, cnt = _search(posp[:, 0], posp[:, 1], posp[:, 2],
                      posb[:, 0], posb[:, 1], posb[:, 2], s)
    outp = _conv(sg, q, W2, b2r)
    out = jnp.where(cnt[:N, None] > 0, outp[:N], jnp.float32(-1e30))
    return (out, pos, batch)


# speedup vs baseline: 9.2974x; 1.0922x over previous
import functools

import jax
import jax.numpy as jnp
from jax import lax
from jax.experimental import pallas as pl
from jax.experimental.pallas import tpu as pltpu
from jax.experimental.pallas import tpu_sc as plsc

N = 10000
D = 128
K = 64
H = 128
R2 = 0.12 * 0.12

NW = 32
QPW = 320
NP = NW * QPW
NV = NP // 16
BQ = 64
PAD_POS = 1e9


def _prep_body(x_ref, pos_ref, w1a_ref, w1b_ref, b1_ref, s_ref, q_ref):
    pw = (pos_ref[:, 0:1] * w1b_ref[0:1, :]
          + pos_ref[:, 1:2] * w1b_ref[1:2, :]
          + pos_ref[:, 2:3] * w1b_ref[2:3, :])
    xw = lax.dot_general(x_ref[...], w1a_ref[...], (((1,), (0,)), ((), ())),
                         preferred_element_type=jnp.float32)
    s_ref[...] = xw + pw
    q_ref[...] = b1_ref[...] - pw


def _prep(xp, posp, w1a, w1b, b1r):
    blk = 256
    return pl.pallas_call(
        _prep_body,
        grid=(NP // blk,),
        in_specs=[
            pl.BlockSpec((blk, D), lambda i: (i, 0)),
            pl.BlockSpec((blk, 3), lambda i: (i, 0)),
            pl.BlockSpec((D, H), lambda i: (0, 0)),
            pl.BlockSpec((3, H), lambda i: (0, 0)),
            pl.BlockSpec((1, H), lambda i: (0, 0)),
        ],
        out_specs=[
            pl.BlockSpec((blk, H), lambda i: (i, 0)),
            pl.BlockSpec((blk, H), lambda i: (i, 0)),
        ],
        out_shape=[
            jax.ShapeDtypeStruct((NP, H), jnp.float32),
            jax.ShapeDtypeStruct((NP, H), jnp.float32),
        ],
    )(xp, posp, w1a, w1b, b1r)


_mesh = plsc.VectorSubcoreMesh(core_axis_name="c", subcore_axis_name="s")


@functools.partial(
    pl.kernel,
    mesh=_mesh,
    out_type=[
        jax.ShapeDtypeStruct((NP * K, H), jnp.float32),
        jax.ShapeDtypeStruct((NP,), jnp.int32),
    ],
    scratch_types=[
        pltpu.VMEM((NP,), jnp.float32),
        pltpu.VMEM((NP,), jnp.float32),
        pltpu.VMEM((NP,), jnp.float32),
        pltpu.VMEM((NP,), jnp.float32),
        pltpu.VMEM((NP + 32,), jnp.float32),
        pltpu.VMEM((NP + 32,), jnp.int32),
        pltpu.VMEM((QPW * K,), jnp.int32),
        pltpu.VMEM((QPW,), jnp.int32),
        pltpu.VMEM((128, H), jnp.float32),
        pltpu.VMEM((128, H), jnp.float32),
        pltpu.SemaphoreType.DMA,
        pltpu.SemaphoreType.DMA,
        pltpu.SemaphoreType.DMA,
        pltpu.SemaphoreType.DMA,
    ],
    compiler_params=pltpu.CompilerParams(needs_layout_passes=False),
)
def _search(posx_hbm, posy_hbm, posz_hbm, posxb_hbm, posyb_hbm, poszb_hbm,
            s_hbm, sg_hbm, cnt_hbm, px, py, pz, sq, cd, ci, nbrs, cnts,
            rows0, rows1, semg0, semg1, semw0, semw1):
    cid = lax.axis_index("c")
    sid = lax.axis_index("s")
    wid = sid * 2 + cid
    q0 = wid * QPW

    pltpu.sync_copy(posx_hbm, px)
    pltpu.sync_copy(posy_hbm, py)
    pltpu.sync_copy(posz_hbm, pz)

    lanes = lax.iota(jnp.int32, 16)
    lane0 = lanes == 0
    inf16 = jnp.full((16,), jnp.inf, jnp.float32)

    def sq4(v4, carry):
        for u in range(4):
            base = (v4 * 4 + u) * 16
            xv = px[pl.ds(base, 16)]
            yv = py[pl.ds(base, 16)]
            zv = pz[pl.ds(base, 16)]
            sq[pl.ds(base, 16)] = (xv * xv + yv * yv) + zv * zv
        return carry

    lax.fori_loop(0, NV // 4, sq4, jnp.int32(0))

    pltpu.sync_copy(posxb_hbm, px)
    pltpu.sync_copy(posyb_hbm, py)
    pltpu.sync_copy(poszb_hbm, pz)

    def per_query(qi, carry):
        q = q0 + qi
        qs = jnp.full((16,), q, jnp.int32)
        qx = plsc.load_gather(px, [qs])
        qy = plsc.load_gather(py, [qs])
        qz = plsc.load_gather(pz, [qs])
        qsq = plsc.load_gather(sq, [qs])

        def scan4(v4, off):
            for u in range(4):
                base = (v4 * 4 + u) * 16
                dot = (qx * px[pl.ds(base, 16)]
                       + qy * py[pl.ds(base, 16)]) + qz * pz[pl.ds(base, 16)]
                d2 = (qsq + sq[pl.ds(base, 16)]) - 2.0 * dot
                m = d2 <= R2
                plsc.store_compressed(cd.at[pl.ds(off, 16)], d2, mask=m)
                plsc.store_compressed(ci.at[pl.ds(off, 16)], base + lanes, mask=m)
                off = off + jnp.sum(m.astype(jnp.int32))
            return off

        C = lax.fori_loop(0, NV // 4, scan4, jnp.int32(0))
        cd[pl.ds(C, 16)] = inf16
        cnt = jnp.minimum(C, K)
        vc = (C + 15) // 16

        plsc.store_scatter(nbrs, [jnp.full((16,), qi * K, jnp.int32)], qs,
                           mask=lane0)

        @pl.when((C >= 1) & (C <= K))
        def _copy_all():
            for t in range(4):
                nbrs[pl.ds(qi * K + t * 16, 16)] = ci[pl.ds(t * 16, 16)]

        def extract(k, _):
            def m1(v, acc):
                return jnp.minimum(acc, cd[pl.ds(v * 16, 16)])

            mv = lax.fori_loop(0, vc, m1, inf16)
            mn = jnp.min(mv)

            def m2(v, acc):
                d = cd[pl.ds(v * 16, 16)]
                p = v * 16 + lanes
                return jnp.minimum(acc, jnp.where(d == mn, p, NP * 4))

            pv = lax.fori_loop(0, vc, m2, jnp.full((16,), NP * 4, jnp.int32))
            p = jnp.min(pv)
            ps = jnp.full((16,), p, jnp.int32)
            nv = plsc.load_gather(ci, [ps])
            plsc.store_scatter(nbrs, [jnp.full((16,), qi * K + k, jnp.int32)],
                               nv, mask=lane0)
            plsc.store_scatter(cd, [ps], inf16, mask=lane0)
            return _

        @pl.when(C > K)
        def _extract_topk():
            lax.fori_loop(0, cnt, extract, jnp.int32(0))

        nbr0 = plsc.load_gather(nbrs, [jnp.full((16,), qi * K, jnp.int32)])
        for t in range(4):
            kvec = t * 16 + lanes
            plsc.store_scatter(nbrs, [qi * K + kvec], nbr0, mask=kvec >= cnt)
        plsc.store_scatter(cnts, [jnp.full((16,), qi, jnp.int32)],
                           jnp.full((16,), cnt, jnp.int32), mask=lane0)
        return carry

    lax.fori_loop(0, QPW, per_query, jnp.int32(0))
    pltpu.sync_copy(cnts, cnt_hbm.at[pl.ds(q0, QPW)])

    row0 = q0 * K

    def gpair(p, carry):
        ch0 = 2 * p
        ga = pltpu.async_copy(
            s_hbm.at[nbrs.at[pl.ds(ch0 * 128, 128)]], rows0, semg0)
        gb = pltpu.async_copy(
            s_hbm.at[nbrs.at[pl.ds((ch0 + 1) * 128, 128)]], rows1, semg1)
        ga.wait()
        wa = pltpu.async_copy(
            rows0, sg_hbm.at[pl.ds(row0 + ch0 * 128, 128)], semw0)
        gb.wait()
        wb = pltpu.async_copy(
            rows1, sg_hbm.at[pl.ds(row0 + (ch0 + 1) * 128, 128)], semw1)
        wa.wait()
        wb.wait()
        return carry

    lax.fori_loop(0, QPW * K // 256, gpair, jnp.int32(0))


def _conv_body(sg_ref, q_ref, w2_ref, b2_ref, o_ref):
    z3 = sg_ref[...].reshape(BQ, K, H) + q_ref[...][:, None, :]
    z = jnp.maximum(z3, 0.0).reshape(BQ * K, H)
    h = lax.dot_general(z, w2_ref[...], (((1,), (0,)), ((), ())),
                        preferred_element_type=jnp.float32) + b2_ref[...]
    o_ref[...] = jnp.max(h.reshape(BQ, K, H), axis=1)


def _conv(sg, q, W2, b2r):
    return pl.pallas_call(
        _conv_body,
        grid=(NP // BQ,),
        in_specs=[
            pl.BlockSpec((BQ * K, H), lambda i: (i, 0)),
            pl.BlockSpec((BQ, H), lambda i: (i, 0)),
            pl.BlockSpec((H, H), lambda i: (0, 0)),
            pl.BlockSpec((1, H), lambda i: (0, 0)),
        ],
        out_specs=pl.BlockSpec((BQ, H), lambda i: (i, 0)),
        out_shape=jax.ShapeDtypeStruct((NP, H), jnp.float32),
    )(sg, q, W2, b2r)


def kernel(x, pos, batch, W1, b1, W2, b2):
    pad = NP - N
    xp = jnp.concatenate([x, jnp.zeros((pad, D), jnp.float32)], axis=0)
    posp = jnp.concatenate(
        [pos, jnp.full((pad, 3), PAD_POS, jnp.float32)], axis=0)
    w1a = W1[:D]
    w1b = W1[D:]
    b1r = b1.reshape(1, H)
    b2r = b2.reshape(1, H)

    posb = lax.reduce_precision(posp, exponent_bits=8, mantissa_bits=7)
    s, q = _prep(xp, posp, w1a, w1b, b1r)
    sg, cnt = _search(posp[:, 0], posp[:, 1], posp[:, 2],
                      posb[:, 0], posb[:, 1], posb[:, 2], s)
    outp = _conv(sg, q, W2, b2r)
    out = jnp.where(cnt[:N, None] > 0, outp[:N], jnp.float32(-1e30))
    return (out, pos, batch)
